# baseline (speedup 1.0000x reference)
import functools

import jax
import jax.numpy as jnp
from jax import lax
from jax.experimental import pallas as pl
from jax.experimental.pallas import tpu as pltpu
from jax.experimental.pallas import tpu_sc as plsc

B, T, D = 1024, 200, 64
N = B * T
NC, NS, L = 2, 16, 16
NW = NC * NS
PER_W = N // NW
CH = 128
G = PER_W // CH
NBUF = 6
LOOK = 4

_mesh = plsc.VectorSubcoreMesh(core_axis_name="c", subcore_axis_name="s")


@functools.partial(
    pl.kernel,
    mesh=_mesh,
    out_type=jax.ShapeDtypeStruct((N, D), jnp.float32),
    scratch_types=[
        pltpu.VMEM((1, G, CH), jnp.int32),
        pltpu.VMEM((1, G, CH), jnp.int32),
        pltpu.VMEM((NBUF, CH, D), jnp.float32),
        pltpu.VMEM((NBUF, CH, D), jnp.float32),
        [pltpu.SemaphoreType.DMA] * NBUF,
        [pltpu.SemaphoreType.DMA] * NBUF,
        [pltpu.SemaphoreType.DMA] * NBUF,
    ],
    compiler_params=pltpu.CompilerParams(use_tc_tiling_on_sc=False),
)
def _sc_embed(idx0_hbm, idx1_hbm, tab_hbm, out_hbm,
              idx0_v, idx1_v, bufa, bufb, ga, gb, gc):
    wid = lax.axis_index("s") * NC + lax.axis_index("c")
    gbase = wid * G
    pltpu.sync_copy(idx0_hbm.at[pl.ds(wid, 1)], idx0_v)
    pltpu.sync_copy(idx1_hbm.at[pl.ds(wid, 1)], idx1_v)

    def gathers(j):
        b = j % NBUF
        pltpu.async_copy(tab_hbm.at[idx0_v.at[0, j]], bufa.at[b], ga[b])
        pltpu.async_copy(tab_hbm.at[idx1_v.at[0, j]], bufb.at[b], gb[b])

    def wait_gathers(j):
        b = j % NBUF
        pltpu.make_async_copy(tab_hbm.at[idx0_v.at[0, j]], bufa.at[b],
                              ga[b]).wait()
        pltpu.make_async_copy(tab_hbm.at[idx1_v.at[0, j]], bufb.at[b],
                              gb[b]).wait()

    def scatter(j):
        b = j % NBUF
        pltpu.async_copy(bufa.at[b], out_hbm.at[pl.ds((gbase + j) * CH, CH)],
                         gc[b])

    def wait_scatter(j):
        b = j % NBUF
        pltpu.make_async_copy(bufa.at[b], out_hbm.at[pl.ds((gbase + j) * CH, CH)],
                              gc[b]).wait()

    def accumulate(j):
        b = j % NBUF

        def addrow(r, carry):
            for c in range(D // L):
                sl = pl.ds(c * L, L)
                plsc.addupdate(bufa.at[b, r, sl], bufb[b, r, sl])
            return carry

        lax.fori_loop(0, CH, addrow, 0, unroll=4)

    for j in range(LOOK):
        gathers(j)
    for j in range(G):
        if j + LOOK < G:
            if j >= NBUF - LOOK:
                wait_scatter(j - (NBUF - LOOK))
            gathers(j + LOOK)
        wait_gathers(j)
        accumulate(j)
        scatter(j)
    for j in range(G - NBUF, G):
        wait_scatter(j)


def kernel(X, w_embed):
    Xf = X.reshape(N, 2).astype(jnp.int32)
    idx0 = Xf[:, 0].reshape(NW, G, CH)
    idx1 = Xf[:, 1].reshape(NW, G, CH)
    h = _sc_embed(idx0, idx1, w_embed)
    return h.reshape(B, T, D), w_embed


# speedup vs baseline: 1.0071x; 1.0071x over previous
import functools

import jax
import jax.numpy as jnp
from jax import lax
from jax.experimental import pallas as pl
from jax.experimental.pallas import tpu as pltpu
from jax.experimental.pallas import tpu_sc as plsc

B, T, D = 1024, 200, 64
N = B * T
NC, NS, L = 2, 16, 16
NW = NC * NS
PER_W = N // NW
CH = 400
G = PER_W // CH
NBUF = 4
LOOK = 2

_mesh = plsc.VectorSubcoreMesh(core_axis_name="c", subcore_axis_name="s")


@functools.partial(
    pl.kernel,
    mesh=_mesh,
    out_type=jax.ShapeDtypeStruct((N, D), jnp.float32),
    scratch_types=[
        pltpu.VMEM((1, G, CH), jnp.int32),
        pltpu.VMEM((1, G, CH), jnp.int32),
        pltpu.VMEM((NBUF, CH, D), jnp.float32),
        [pltpu.SemaphoreType.DMA] * NBUF,
        [pltpu.SemaphoreType.DMA] * NBUF,
        [pltpu.SemaphoreType.DMA] * NBUF,
    ],
    compiler_params=pltpu.CompilerParams(use_tc_tiling_on_sc=False),
)
def _sc_embed(idx0_hbm, idx1_hbm, tab_hbm, out_hbm,
              idx0_v, idx1_v, buf, ga, gb, gc):
    wid = lax.axis_index("s") * NC + lax.axis_index("c")
    gbase = wid * G
    pltpu.sync_copy(idx0_hbm.at[pl.ds(wid, 1)], idx0_v)
    pltpu.sync_copy(idx1_hbm.at[pl.ds(wid, 1)], idx1_v)

    def gather(j, idx_v, sem, add):
        b = j % NBUF
        pltpu.async_copy(tab_hbm.at[idx_v.at[0, j]], buf.at[b], sem[b],
                         add=add)

    def wait_gather(j, idx_v, sem):
        b = j % NBUF
        pltpu.make_async_copy(tab_hbm.at[idx_v.at[0, j]], buf.at[b],
                              sem[b]).wait()

    def scatter(j):
        b = j % NBUF
        pltpu.async_copy(buf.at[b], out_hbm.at[pl.ds((gbase + j) * CH, CH)],
                         gc[b])

    def wait_scatter(j):
        b = j % NBUF
        pltpu.make_async_copy(buf.at[b], out_hbm.at[pl.ds((gbase + j) * CH, CH)],
                              gc[b]).wait()

    for j in range(LOOK):
        gather(j, idx0_v, ga, False)
    for j in range(G):
        if j + LOOK < G:
            if j >= NBUF - LOOK:
                wait_scatter(j - (NBUF - LOOK))
            gather(j + LOOK, idx0_v, ga, False)
        wait_gather(j, idx0_v, ga)
        gather(j, idx1_v, gb, True)
        wait_gather(j, idx1_v, gb)
        scatter(j)
    for j in range(G - NBUF, G):
        wait_scatter(j)


def kernel(X, w_embed):
    Xf = X.reshape(N, 2).astype(jnp.int32)
    idx0 = Xf[:, 0].reshape(NW, G, CH)
    idx1 = Xf[:, 1].reshape(NW, G, CH)
    h = _sc_embed(idx0, idx1, w_embed)
    return h.reshape(B, T, D), w_embed
